# baseline (speedup 1.0000x reference)
import functools
import math

import jax
import jax.numpy as jnp
import numpy as np
from jax.experimental import pallas as pl
from jax.experimental.pallas import tpu as pltpu

K = 16
O = 8
N = 384
NF = 193
NFP = 256


def _dft_constants():
    n = np.arange(N, dtype=np.int64)
    k = np.arange(NFP, dtype=np.int64)
    ang = -2.0 * np.pi * ((n[:, None] * k[None, :]) % N).astype(np.float64) / N
    scale = 1.0 / N
    wr = np.cos(ang) * scale
    wi = np.sin(ang) * scale
    wr[:, NF:] = 0.0
    wi[:, NF:] = 0.0
    m = np.arange(N, dtype=np.int64)
    ang2 = -2.0 * np.pi * ((n[:, None] * m[None, :]) % N).astype(np.float64) / N
    fr = np.cos(ang2)
    fi = np.sin(ang2)
    return (wr.astype(np.float32), wi.astype(np.float32),
            fr.astype(np.float32), fi.astype(np.float32),
            (fr + fi).astype(np.float32))


_BF16 = jnp.bfloat16


_WR, _WI, _FR, _FI, _FRPI = _dft_constants()


def _spec_kernel(x_ref, wrwi_ref, ff_ref, out_ref, *, cpb):
    c = pl.program_id(1)
    a_parts = []
    for j in range(cpb):
        xi = x_ref[0, j].astype(_BF16)
        aa = jnp.dot(xi, wrwi_ref[...], preferred_element_type=jnp.float32)
        a_parts.append(aa.astype(_BF16))
    a_cat = jnp.concatenate(a_parts, axis=1)
    tt = jnp.dot(ff_ref[...], a_cat, preferred_element_type=jnp.float32)
    msum = None
    for j in range(cpb):
        rr_ri = tt[:N, j * 2 * NFP:(j + 1) * 2 * NFP]
        ii_ri = tt[N:, j * 2 * NFP:(j + 1) * 2 * NFP]
        yr = rr_ri[:, :NFP] - ii_ri[:, NFP:]
        yi = rr_ri[:, NFP:] + ii_ri[:, :NFP]
        mag = jnp.sqrt(yr * yr + yi * yi)
        msum = mag if msum is None else msum + mag

    @pl.when(c == 0)
    def _():
        out_ref[0] = msum

    @pl.when(c != 0)
    def _():
        out_ref[0] = out_ref[0] + msum


def _hist_kernel(spec_ref, rb_ref, ob_ref, wt_ref, bvec_ref, out_ref, *, inv_c):
    mag = jnp.log1p(spec_ref[...] * inv_c)
    rb = rb_ref[...]
    ob = ob_ref[...]
    cols = []
    for s in range(K):
        cols.append(jnp.sum(jnp.where(rb == s, mag, 0.0), axis=1, keepdims=True))
    for s in range(O):
        cols.append(jnp.sum(jnp.where(ob == s, mag, 0.0), axis=1, keepdims=True))
    h = jnp.concatenate(cols, axis=1)
    rs = jnp.sum(h[:, :K], axis=1, keepdims=True) + 1e-6
    osum = jnp.sum(h[:, K:], axis=1, keepdims=True) + 1e-6
    col = jax.lax.broadcasted_iota(jnp.int32, (h.shape[0], K + O), 1)
    hn = h / jnp.where(col < K, rs, osum)
    out_ref[...] = (jnp.dot(hn, wt_ref[...], preferred_element_type=jnp.float32)
                    + bvec_ref[...])


def _bin_ids():
    yy, xx = jnp.meshgrid(jnp.linspace(-1.0, 1.0, N),
                          jnp.linspace(0.0, 1.0, NF), indexing='ij')
    rr = jnp.clip(jnp.sqrt(yy ** 2 + xx ** 2), 0.0, 1.0 - 1e-08)
    th = jnp.arctan2(yy, xx + 1e-09) + math.pi / 2
    rb = jnp.clip((rr * K).astype(jnp.int32), 0, K - 1)
    ob = jnp.clip((th / math.pi * O).astype(jnp.int32), 0, O - 1)
    rb = jnp.pad(rb, ((0, 0), (0, NFP - NF)), constant_values=-1)
    ob = jnp.pad(ob, ((0, 0), (0, NFP - NF)), constant_values=-1)
    return rb.reshape(1, N * NFP), ob.reshape(1, N * NFP)


def kernel(x, W, b):
    B, C = x.shape[0], x.shape[1]

    cpb = 8 if C % 8 == 0 else 1
    wrwi = jnp.concatenate([jnp.asarray(_WR, dtype=_BF16),
                            jnp.asarray(_WI, dtype=_BF16)], axis=1)
    spec = pl.pallas_call(
        functools.partial(_spec_kernel, cpb=cpb),
        grid=(B, C // cpb),
        in_specs=[
            pl.BlockSpec((1, cpb, N, N), lambda bb, cc: (bb, cc, 0, 0)),
            pl.BlockSpec((N, 2 * NFP), lambda bb, cc: (0, 0)),
            pl.BlockSpec((2 * N, N), lambda bb, cc: (0, 0)),
        ],
        out_specs=pl.BlockSpec((1, N, NFP), lambda bb, cc: (bb, 0, 0)),
        out_shape=jax.ShapeDtypeStruct((B, N, NFP), jnp.float32),
        compiler_params=pltpu.CompilerParams(
            dimension_semantics=("parallel", "arbitrary")),
    )(x, wrwi, jnp.concatenate([jnp.asarray(_FR, dtype=_BF16),
                                jnp.asarray(_FI, dtype=_BF16)], axis=0))

    rb, ob = _bin_ids()
    out = pl.pallas_call(
        functools.partial(_hist_kernel, inv_c=1.0 / C),
        out_shape=jax.ShapeDtypeStruct((B, W.shape[0]), jnp.float32),
    )(spec.reshape(B, N * NFP), rb, ob, W.T, b.reshape(1, -1))
    return out


# speedup vs baseline: 1.0765x; 1.0765x over previous
import functools
import math

import jax
import jax.numpy as jnp
import numpy as np
from jax.experimental import pallas as pl
from jax.experimental.pallas import tpu as pltpu

K = 16
O = 8
N = 384
NF = 193
NFP = 256


def _dft_constants():
    n = np.arange(N, dtype=np.int64)
    k = np.arange(NFP, dtype=np.int64)
    ang = -2.0 * np.pi * ((n[:, None] * k[None, :]) % N).astype(np.float64) / N
    scale = 1.0 / N
    wr = np.cos(ang) * scale
    wi = np.sin(ang) * scale
    wr[:, NF:] = 0.0
    wi[:, NF:] = 0.0
    h = N // 2
    n2 = np.arange(h, dtype=np.int64)
    ang2 = -2.0 * np.pi * ((n2[:, None] * n2[None, :]) % h).astype(np.float64) / h
    fr = np.cos(ang2)
    fi = np.sin(ang2)
    tw = -2.0 * np.pi * n2.astype(np.float64) / N
    twr = np.broadcast_to(np.cos(tw)[:, None], (h, NFP)).copy()
    twi = np.broadcast_to(np.sin(tw)[:, None], (h, NFP)).copy()
    return (wr.astype(np.float32), wi.astype(np.float32),
            fr.astype(np.float32), fi.astype(np.float32),
            (fr + fi).astype(np.float32),
            twr.astype(np.float32), twi.astype(np.float32))


_BF16 = jnp.bfloat16


_WR, _WI, _FR, _FI, _FRPI, _TWR, _TWI = _dft_constants()
_H = N // 2


def _spec_kernel(x_ref, wrwi_ref, fr_ref, fi_ref, frpi_ref, twr_ref, twi_ref,
                 out_ref, *, cpb):
    c = pl.program_id(1)
    twr = twr_ref[...]
    twi = twi_ref[...]
    r_parts, i_parts, s_parts = [], [], []
    for j in range(cpb):
        xi = x_ref[0, j].astype(_BF16)
        aa = jnp.dot(xi, wrwi_ref[...], preferred_element_type=jnp.float32)
        u_r = aa[:_H, :NFP] + aa[_H:, :NFP]
        u_i = aa[:_H, NFP:] + aa[_H:, NFP:]
        d_r = aa[:_H, :NFP] - aa[_H:, :NFP]
        d_i = aa[:_H, NFP:] - aa[_H:, NFP:]
        v_r = d_r * twr - d_i * twi
        v_i = d_r * twi + d_i * twr
        r_parts.append(jnp.concatenate([u_r, v_r], axis=1).astype(_BF16))
        i_parts.append(jnp.concatenate([u_i, v_i], axis=1).astype(_BF16))
        s_parts.append(jnp.concatenate([u_r + u_i, v_r + v_i],
                                       axis=1).astype(_BF16))
    r_cat = jnp.concatenate(r_parts, axis=1)
    i_cat = jnp.concatenate(i_parts, axis=1)
    s_cat = jnp.concatenate(s_parts, axis=1)
    t1 = jnp.dot(fr_ref[...], r_cat, preferred_element_type=jnp.float32)
    t2 = jnp.dot(fi_ref[...], i_cat, preferred_element_type=jnp.float32)
    t3 = jnp.dot(frpi_ref[...], s_cat, preferred_element_type=jnp.float32)
    yr = t1 - t2
    yi = t3 - t1 - t2
    mag = jnp.sqrt(yr * yr + yi * yi)
    me = mag[:, :NFP]
    mo = mag[:, NFP:2 * NFP]
    for j in range(1, cpb):
        me = me + mag[:, j * 2 * NFP:j * 2 * NFP + NFP]
        mo = mo + mag[:, j * 2 * NFP + NFP:(j + 1) * 2 * NFP]
    msum = jnp.concatenate([me, mo], axis=0)

    @pl.when(c == 0)
    def _():
        out_ref[0] = msum

    @pl.when(c != 0)
    def _():
        out_ref[0] = out_ref[0] + msum


def _hist_kernel(spec_ref, rb_ref, ob_ref, wt_ref, bvec_ref, out_ref, *, inv_c):
    mag = jnp.log1p(spec_ref[...] * inv_c)
    rb = rb_ref[...]
    ob = ob_ref[...]
    cols = []
    for s in range(K):
        cols.append(jnp.sum(jnp.where(rb == s, mag, 0.0), axis=1, keepdims=True))
    for s in range(O):
        cols.append(jnp.sum(jnp.where(ob == s, mag, 0.0), axis=1, keepdims=True))
    h = jnp.concatenate(cols, axis=1)
    rs = jnp.sum(h[:, :K], axis=1, keepdims=True) + 1e-6
    osum = jnp.sum(h[:, K:], axis=1, keepdims=True) + 1e-6
    col = jax.lax.broadcasted_iota(jnp.int32, (h.shape[0], K + O), 1)
    hn = h / jnp.where(col < K, rs, osum)
    out_ref[...] = (jnp.dot(hn, wt_ref[...], preferred_element_type=jnp.float32)
                    + bvec_ref[...])


def _bin_ids():
    yy, xx = jnp.meshgrid(jnp.linspace(-1.0, 1.0, N),
                          jnp.linspace(0.0, 1.0, NF), indexing='ij')
    rr = jnp.clip(jnp.sqrt(yy ** 2 + xx ** 2), 0.0, 1.0 - 1e-08)
    th = jnp.arctan2(yy, xx + 1e-09) + math.pi / 2
    rb = jnp.clip((rr * K).astype(jnp.int32), 0, K - 1)
    ob = jnp.clip((th / math.pi * O).astype(jnp.int32), 0, O - 1)
    rb = jnp.pad(rb, ((0, 0), (0, NFP - NF)), constant_values=-1)
    ob = jnp.pad(ob, ((0, 0), (0, NFP - NF)), constant_values=-1)
    row_perm = jnp.concatenate([jnp.arange(0, N, 2), jnp.arange(1, N, 2)])
    rb = rb[row_perm]
    ob = ob[row_perm]
    return rb.reshape(1, N * NFP), ob.reshape(1, N * NFP)


def kernel(x, W, b):
    B, C = x.shape[0], x.shape[1]

    cpb = 8 if C % 8 == 0 else 1
    wrwi = jnp.concatenate([jnp.asarray(_WR, dtype=_BF16),
                            jnp.asarray(_WI, dtype=_BF16)], axis=1)
    spec = pl.pallas_call(
        functools.partial(_spec_kernel, cpb=cpb),
        grid=(B, C // cpb),
        in_specs=[
            pl.BlockSpec((1, cpb, N, N), lambda bb, cc: (bb, cc, 0, 0)),
            pl.BlockSpec((N, 2 * NFP), lambda bb, cc: (0, 0)),
            pl.BlockSpec((_H, _H), lambda bb, cc: (0, 0)),
            pl.BlockSpec((_H, _H), lambda bb, cc: (0, 0)),
            pl.BlockSpec((_H, _H), lambda bb, cc: (0, 0)),
            pl.BlockSpec((_H, NFP), lambda bb, cc: (0, 0)),
            pl.BlockSpec((_H, NFP), lambda bb, cc: (0, 0)),
        ],
        out_specs=pl.BlockSpec((1, N, NFP), lambda bb, cc: (bb, 0, 0)),
        out_shape=jax.ShapeDtypeStruct((B, N, NFP), jnp.float32),
        compiler_params=pltpu.CompilerParams(
            dimension_semantics=("parallel", "arbitrary")),
    )(x, wrwi, jnp.asarray(_FR, dtype=_BF16), jnp.asarray(_FI, dtype=_BF16),
      jnp.asarray(_FRPI, dtype=_BF16), jnp.asarray(_TWR), jnp.asarray(_TWI))

    rb, ob = _bin_ids()
    out = pl.pallas_call(
        functools.partial(_hist_kernel, inv_c=1.0 / C),
        out_shape=jax.ShapeDtypeStruct((B, W.shape[0]), jnp.float32),
    )(spec.reshape(B, N * NFP), rb, ob, W.T, b.reshape(1, -1))
    return out


# speedup vs baseline: 1.2520x; 1.1630x over previous
import functools
import math

import jax
import jax.numpy as jnp
import numpy as np
from jax.experimental import pallas as pl
from jax.experimental.pallas import tpu as pltpu

K = 16
O = 8
N = 384
NF = 193
NFP = 256


def _dft_constants():
    n = np.arange(N, dtype=np.int64)
    k = np.arange(NFP, dtype=np.int64)
    ang = -2.0 * np.pi * ((n[:, None] * k[None, :]) % N).astype(np.float64) / N
    scale = 1.0 / N
    wr = np.cos(ang) * scale
    wi = np.sin(ang) * scale
    wr[:, NF:] = 0.0
    wi[:, NF:] = 0.0
    h = N // 2
    n2 = np.arange(h, dtype=np.int64)
    ang2 = -2.0 * np.pi * ((n2[:, None] * n2[None, :]) % h).astype(np.float64) / h
    fr = np.cos(ang2)
    fi = np.sin(ang2)
    tw = -2.0 * np.pi * n2.astype(np.float64) / N
    twr, twi = np.cos(tw)[None, :], np.sin(tw)[None, :]
    fwr = fr * twr - fi * twi
    fwi = fr * twi + fi * twr
    return (wr.astype(np.float32), wi.astype(np.float32),
            fr.astype(np.float32), fi.astype(np.float32),
            (fr + fi).astype(np.float32),
            fwr.astype(np.float32), fwi.astype(np.float32),
            (fwr + fwi).astype(np.float32))


_BF16 = jnp.bfloat16


_WR, _WI, _FR, _FI, _FRPI, _FWR, _FWI, _FWRPI = _dft_constants()
_H = N // 2


def _spec_kernel(x_ref, wrwi_ref, fr_ref, fi_ref, frpi_ref, fwr_ref, fwi_ref,
                 fwrpi_ref, out_ref, *, cpb):
    c = pl.program_id(1)
    ru, iu, su, rd, sd, id_ = [], [], [], [], [], []
    for j in range(cpb):
        xi = x_ref[0, j].astype(_BF16)
        aa = jnp.dot(xi, wrwi_ref[...],
                     preferred_element_type=jnp.float32).astype(_BF16)
        u_r = aa[:_H, :NFP] + aa[_H:, :NFP]
        u_i = aa[:_H, NFP:] + aa[_H:, NFP:]
        d_r = aa[:_H, :NFP] - aa[_H:, :NFP]
        d_i = aa[:_H, NFP:] - aa[_H:, NFP:]
        ru.append(u_r)
        iu.append(u_i)
        su.append(u_r + u_i)
        rd.append(d_r)
        id_.append(d_i)
        sd.append(d_r + d_i)
    ru_cat = jnp.concatenate(ru, axis=1)
    iu_cat = jnp.concatenate(iu, axis=1)
    su_cat = jnp.concatenate(su, axis=1)
    rd_cat = jnp.concatenate(rd, axis=1)
    id_cat = jnp.concatenate(id_, axis=1)
    sd_cat = jnp.concatenate(sd, axis=1)
    te1 = jnp.dot(fr_ref[...], ru_cat, preferred_element_type=jnp.float32)
    te2 = jnp.dot(fi_ref[...], iu_cat, preferred_element_type=jnp.float32)
    te3 = jnp.dot(frpi_ref[...], su_cat, preferred_element_type=jnp.float32)
    to1 = jnp.dot(fwr_ref[...], rd_cat, preferred_element_type=jnp.float32)
    to2 = jnp.dot(fwi_ref[...], id_cat, preferred_element_type=jnp.float32)
    to3 = jnp.dot(fwrpi_ref[...], sd_cat, preferred_element_type=jnp.float32)
    yre = te1 - te2
    yie = te3 - te1 - te2
    mage = jnp.sqrt(yre * yre + yie * yie)
    yro = to1 - to2
    yio = to3 - to1 - to2
    mago = jnp.sqrt(yro * yro + yio * yio)
    me = mage[:, :NFP]
    mo = mago[:, :NFP]
    for j in range(1, cpb):
        me = me + mage[:, j * NFP:(j + 1) * NFP]
        mo = mo + mago[:, j * NFP:(j + 1) * NFP]
    msum = jnp.concatenate([me, mo], axis=0)

    @pl.when(c == 0)
    def _():
        out_ref[0] = msum

    @pl.when(c != 0)
    def _():
        out_ref[0] = out_ref[0] + msum


def _hist_kernel(spec_ref, rb_ref, ob_ref, wt_ref, bvec_ref, out_ref, *, inv_c):
    mag = jnp.log1p(spec_ref[...] * inv_c)
    rb = rb_ref[...]
    ob = ob_ref[...]
    cols = []
    for s in range(K):
        cols.append(jnp.sum(jnp.where(rb == s, mag, 0.0), axis=1, keepdims=True))
    for s in range(O):
        cols.append(jnp.sum(jnp.where(ob == s, mag, 0.0), axis=1, keepdims=True))
    h = jnp.concatenate(cols, axis=1)
    rs = jnp.sum(h[:, :K], axis=1, keepdims=True) + 1e-6
    osum = jnp.sum(h[:, K:], axis=1, keepdims=True) + 1e-6
    col = jax.lax.broadcasted_iota(jnp.int32, (h.shape[0], K + O), 1)
    hn = h / jnp.where(col < K, rs, osum)
    out_ref[...] = (jnp.dot(hn, wt_ref[...], preferred_element_type=jnp.float32)
                    + bvec_ref[...])


def _bin_ids():
    yy, xx = jnp.meshgrid(jnp.linspace(-1.0, 1.0, N),
                          jnp.linspace(0.0, 1.0, NF), indexing='ij')
    rr = jnp.clip(jnp.sqrt(yy ** 2 + xx ** 2), 0.0, 1.0 - 1e-08)
    th = jnp.arctan2(yy, xx + 1e-09) + math.pi / 2
    rb = jnp.clip((rr * K).astype(jnp.int32), 0, K - 1)
    ob = jnp.clip((th / math.pi * O).astype(jnp.int32), 0, O - 1)
    rb = jnp.pad(rb, ((0, 0), (0, NFP - NF)), constant_values=-1)
    ob = jnp.pad(ob, ((0, 0), (0, NFP - NF)), constant_values=-1)
    row_perm = jnp.concatenate([jnp.arange(0, N, 2), jnp.arange(1, N, 2)])
    rb = rb[row_perm]
    ob = ob[row_perm]
    return rb.reshape(1, N * NFP), ob.reshape(1, N * NFP)


def kernel(x, W, b):
    B, C = x.shape[0], x.shape[1]

    cpb = 8 if C % 8 == 0 else 1
    wrwi = jnp.concatenate([jnp.asarray(_WR, dtype=_BF16),
                            jnp.asarray(_WI, dtype=_BF16)], axis=1)
    spec = pl.pallas_call(
        functools.partial(_spec_kernel, cpb=cpb),
        grid=(B, C // cpb),
        in_specs=[
            pl.BlockSpec((1, cpb, N, N), lambda bb, cc: (bb, cc, 0, 0)),
            pl.BlockSpec((N, 2 * NFP), lambda bb, cc: (0, 0)),
            pl.BlockSpec((_H, _H), lambda bb, cc: (0, 0)),
            pl.BlockSpec((_H, _H), lambda bb, cc: (0, 0)),
            pl.BlockSpec((_H, _H), lambda bb, cc: (0, 0)),
            pl.BlockSpec((_H, _H), lambda bb, cc: (0, 0)),
            pl.BlockSpec((_H, _H), lambda bb, cc: (0, 0)),
            pl.BlockSpec((_H, _H), lambda bb, cc: (0, 0)),
        ],
        out_specs=pl.BlockSpec((1, N, NFP), lambda bb, cc: (bb, 0, 0)),
        out_shape=jax.ShapeDtypeStruct((B, N, NFP), jnp.float32),
        compiler_params=pltpu.CompilerParams(
            dimension_semantics=("parallel", "arbitrary")),
    )(x, wrwi, jnp.asarray(_FR, dtype=_BF16), jnp.asarray(_FI, dtype=_BF16),
      jnp.asarray(_FRPI, dtype=_BF16), jnp.asarray(_FWR, dtype=_BF16),
      jnp.asarray(_FWI, dtype=_BF16), jnp.asarray(_FWRPI, dtype=_BF16))

    rb, ob = _bin_ids()
    out = pl.pallas_call(
        functools.partial(_hist_kernel, inv_c=1.0 / C),
        out_shape=jax.ShapeDtypeStruct((B, W.shape[0]), jnp.float32),
    )(spec.reshape(B, N * NFP), rb, ob, W.T, b.reshape(1, -1))
    return out
